# Initial kernel scaffold; baseline (speedup 1.0000x reference)
import jax
import jax.numpy as jnp
from jax.experimental import pallas as pl


def kernel(indices, table):
    raise NotImplementedError("write your pallas kernel here")


# speedup vs baseline: 4.0275x; 4.0275x over previous
import functools

import jax
import jax.numpy as jnp
from jax import lax
from jax.experimental import pallas as pl
from jax.experimental.pallas import tpu as pltpu
from jax.experimental.pallas import tpu_sc as plsc

NUM_WORKERS = 32
CHUNK = 128
SUP = 2048
N_CH = SUP // CHUNK


@functools.partial(jax.jit, static_argnums=(2, 3))
def _gather_flat(idx, table, total, d):
    b_per_w = total // NUM_WORKERS
    n_sup = b_per_w // SUP

    mesh = plsc.VectorSubcoreMesh(core_axis_name="c", subcore_axis_name="s")

    @functools.partial(
        pl.kernel,
        mesh=mesh,
        out_type=jax.ShapeDtypeStruct((total, d), jnp.float32),
        scratch_types=[
            pltpu.VMEM((N_CH, CHUNK), jnp.int32),
            pltpu.VMEM((CHUNK, d), jnp.float32),
            pltpu.SemaphoreType.DMA,
        ],
        compiler_params=pltpu.CompilerParams(use_tc_tiling_on_sc=False),
    )
    def k(idx_hbm, table_hbm, out_hbm, idx_v, rows_v, sem):
        wid = lax.axis_index("s") * 2 + lax.axis_index("c")
        base = wid * b_per_w

        def sup_body(s, carry):
            row0 = base + s * SUP
            off = pl.multiple_of(row0 // CHUNK, 8)
            pltpu.sync_copy(idx_hbm.at[pl.ds(off, N_CH)], idx_v)

            def ch_body(c, carry2):
                pltpu.async_copy(table_hbm.at[idx_v.at[c]], rows_v, sem).wait()
                pltpu.sync_copy(rows_v, out_hbm.at[pl.ds(row0 + c * CHUNK, CHUNK)])
                return carry2

            return lax.fori_loop(0, N_CH, ch_body, carry)

        lax.fori_loop(0, n_sup, sup_body, 0)

    return k(idx.reshape(total // CHUNK, CHUNK), table)


def kernel(indices, table):
    b, h = indices.shape
    v, d = table.shape
    total = b * h
    idx = indices.reshape(total).astype(jnp.int32)
    out = _gather_flat(idx, table, total, d)
    return out.reshape(b, h, d)


# speedup vs baseline: 5.0332x; 1.2497x over previous
import functools

import jax
import jax.numpy as jnp
from jax import lax
from jax.experimental import pallas as pl
from jax.experimental.pallas import tpu as pltpu
from jax.experimental.pallas import tpu_sc as plsc

NUM_WORKERS = 32
CHUNK = 128
K = 8
SUP = K * CHUNK


@functools.partial(jax.jit, static_argnums=(2, 3))
def _gather_flat(idx, table, total, d):
    b_per_w = total // NUM_WORKERS
    n_sup = b_per_w // SUP

    mesh = plsc.VectorSubcoreMesh(core_axis_name="c", subcore_axis_name="s")

    @functools.partial(
        pl.kernel,
        mesh=mesh,
        out_type=jax.ShapeDtypeStruct((total, d), jnp.float32),
        scratch_types=[
            pltpu.VMEM((2, K, CHUNK), jnp.int32),
            pltpu.VMEM((2, SUP, d), jnp.float32),
            pltpu.SemaphoreType.DMA,
            pltpu.SemaphoreType.DMA,
            pltpu.SemaphoreType.DMA,
            pltpu.SemaphoreType.DMA,
            pltpu.SemaphoreType.DMA,
        ],
        compiler_params=pltpu.CompilerParams(use_tc_tiling_on_sc=False),
    )
    def k(idx_hbm, table_hbm, out_hbm, idx_v, rows_v, i_sem0, i_sem1,
          g_sem, o_sem0, o_sem1):
        wid = lax.axis_index("s") * 2 + lax.axis_index("c")
        base = wid * b_per_w
        ibase = base // CHUNK
        i_sems = (i_sem0, i_sem1)
        o_sems = (o_sem0, o_sem1)

        def prefetch_idx(s, p):
            off = pl.multiple_of(ibase + s * K, 8)
            pltpu.async_copy(idx_hbm.at[pl.ds(off, K)], idx_v.at[p], i_sems[p])

        def do_block(s, p, first):
            pltpu.make_async_copy(idx_hbm.at[pl.ds(0, K)], idx_v.at[p],
                                  i_sems[p]).wait()
            if not first:
                pltpu.make_async_copy(out_hbm.at[pl.ds(0, SUP)], rows_v.at[p],
                                      o_sems[p]).wait()
            handles = [
                pltpu.async_copy(table_hbm.at[idx_v.at[p, b]],
                                 rows_v.at[p, pl.ds(b * CHUNK, CHUNK)], g_sem)
                for b in range(K)
            ]
            for h in handles:
                h.wait()
            prefetch_idx(jnp.minimum(s + 2, n_sup - 1), p)
            pltpu.async_copy(rows_v.at[p],
                             out_hbm.at[pl.ds(base + s * SUP, SUP)], o_sems[p])

        prefetch_idx(0, 0)
        prefetch_idx(1, 1)
        do_block(0, 0, first=True)
        do_block(1, 1, first=True)

        def body(g, carry):
            do_block(2 * g, 0, first=False)
            do_block(2 * g + 1, 1, first=False)
            return carry

        lax.fori_loop(1, n_sup // 2, body, 0)

        for p in range(2):
            pltpu.make_async_copy(idx_hbm.at[pl.ds(0, K)], idx_v.at[p],
                                  i_sems[p]).wait()
            pltpu.make_async_copy(out_hbm.at[pl.ds(0, SUP)], rows_v.at[p],
                                  o_sems[p]).wait()

    return k(idx.reshape(total // CHUNK, CHUNK), table)


def kernel(indices, table):
    b, h = indices.shape
    v, d = table.shape
    total = b * h
    idx = indices.reshape(total).astype(jnp.int32)
    out = _gather_flat(idx, table, total, d)
    return out.reshape(b, h, d)
